# baseline (speedup 1.0000x reference)
import jax
import jax.numpy as jnp
from jax import lax
from jax.experimental import pallas as pl
from jax.experimental.pallas import tpu as pltpu, tpu_sc as plsc

Q_TOTAL = 65536
N_SRC = 100000
K_NB = 64
SIGMA_N_CONST = 0.8
EPS_CONST = 1e-08
NUM_TILES = 32
NCHUNKS_Q = 2
QC = Q_TOTAL // NCHUNKS_Q
QPT = QC // NUM_TILES
BLK_Q = 16
NBLK = QPT // BLK_Q
ROWS_PER_BLK = BLK_Q * K_NB
IDX_CHUNK = 128
NCHUNK = ROWS_PER_BLK // IDX_CHUNK
QG = QC // 128
GPT = QPT // 128
UNROLL = 2

_NEG_INV_SIG = -1.0 / (SIGMA_N_CONST * SIGMA_N_CONST + EPS_CONST)


def _sc_body(packed_hbm, qx_hbm, qy_hbm, qz_hbm, h_hbm, idx_hbm, d_hbm,
             idx_v, rows_v, der_v, q_v, gsem, isem, dsem):
    nc = 2
    wid = lax.axis_index("s") * nc + lax.axis_index("c")
    qbase = pl.multiple_of(wid * QPT, QPT)

    for i, src_h in enumerate((qx_hbm, qy_hbm, qz_hbm, h_hbm)):
        pltpu.sync_copy(src_h.at[pl.ds(qbase, QPT)], q_v.at[i])

    lane = lax.broadcasted_iota(jnp.int32, (16,), 0)
    lane64 = lane * K_NB
    cols = [jnp.full((16,), c, jnp.int32) for c in range(6)]
    nmax_u = jnp.full((16,), N_SRC - 1, jnp.uint32)

    def idx_slice(b):
        g0 = qbase + b * BLK_Q
        irow = pl.multiple_of(g0 // 2, NCHUNK)
        return idx_hbm.at[pl.ds(irow, NCHUNK)]

    def fire_idx(b, par):
        pltpu.async_copy(idx_slice(b), idx_v.at[par], isem)

    def wait_idx(b, par):
        pltpu.make_async_copy(idx_slice(b), idx_v.at[par], isem).wait()

    def fire_rows(b, par):
        for r in range(NCHUNK):
            for c in range(IDX_CHUNK // 16):
                sl = (par, r, pl.ds(c * 16, 16))
                v = plsc.bitcast(idx_v[sl], jnp.uint32)
                idx_v[sl] = plsc.bitcast(jnp.minimum(v, nmax_u), jnp.int32)
        for j in range(NCHUNK):
            pltpu.async_copy(
                packed_hbm.at[idx_v.at[par, j]],
                rows_v.at[par, pl.ds(j * IDX_CHUNK, IDX_CHUNK)],
                gsem,
            )

    def wait_rows(par):
        for j in range(NCHUNK):
            pltpu.make_async_copy(
                packed_hbm.at[idx_v.at[par, j]],
                rows_v.at[par, pl.ds(j * IDX_CHUNK, IDX_CHUNK)],
                gsem,
            ).wait()

    def d_slice(b):
        g = wid * GPT + b // 8
        qoff = pl.multiple_of((b % 8) * BLK_Q, BLK_Q)
        return d_hbm.at[g, :, pl.ds(qoff, BLK_Q)]

    def wait_der(b, par):
        pltpu.make_async_copy(der_v.at[par], d_slice(b), dsem).wait()

    fire_idx(0, 0)
    wait_idx(0, 0)
    fire_rows(0, 0)
    fire_idx(1, 1)

    def block_body(b, _):
        par = lax.rem(b, 2)
        nxt = 1 - par
        rows_b = rows_v.at[par]
        der_b = der_v.at[par]

        @pl.when(b + 1 < NBLK)
        def _prefetch_rows():
            wait_idx(b + 1, nxt)
            fire_rows(b + 1, nxt)

        wait_rows(par)

        @pl.when(b + 2 < NBLK)
        def _prefetch_idx():
            fire_idx(b + 2, par)

        @pl.when(b >= 2)
        def _drain_der():
            wait_der(b - 2, par)

        qsl = pl.ds(b * BLK_Q, BLK_Q)
        qx = q_v[0, qsl]
        qy = q_v[1, qsl]
        qz = q_v[2, qsl]
        h = q_v[3, qsl]
        neg_inv_h2 = -1.0 / (h * h + EPS_CONST)
        two_inv = 2.0 * neg_inv_h2

        def base_k(k, carry):
            row = lane64 + k
            nbx = plsc.load_gather(rows_b, [row, cols[0]])
            nby = plsc.load_gather(rows_b, [row, cols[1]])
            nbz = plsc.load_gather(rows_b, [row, cols[2]])
            nnx = plsc.load_gather(rows_b, [row, cols[3]])
            nny = plsc.load_gather(rows_b, [row, cols[4]])
            nnz = plsc.load_gather(rows_b, [row, cols[5]])
            dx = qx - nbx
            dy = qy - nby
            dz = qz - nbz
            d2 = dx * dx + dy * dy + dz * dz
            bw = jnp.exp(d2 * neg_inv_h2)
            f = dx * nnx + dy * nny + dz * nnz
            s = bw * two_inv
            der_b[k, :] = bw
            der_b[K_NB + k, :] = f
            der_b[2 * K_NB + k, :] = s * dx
            der_b[3 * K_NB + k, :] = s * dy
            der_b[4 * K_NB + k, :] = s * dz
            der_b[5 * K_NB + k, :] = nnx
            der_b[6 * K_NB + k, :] = nny
            der_b[7 * K_NB + k, :] = nnz
            return carry

        plsc.parallel_loop(0, K_NB, 1, unroll=UNROLL,
                           carry=jnp.int32(0))(base_k)

        pltpu.async_copy(der_b, d_slice(b), dsem)
        return _

    lax.fori_loop(0, NBLK, block_body, 0)

    wait_der(NBLK - 2, (NBLK - 2) % 2)
    wait_der(NBLK - 1, (NBLK - 1) % 2)


def _tc_body(d_ref, o_ref):
    d = d_ref[0]
    bw = d[0:K_NB, :]
    f = d[K_NB:2 * K_NB, :]
    bgx = d[2 * K_NB:3 * K_NB, :]
    bgy = d[3 * K_NB:4 * K_NB, :]
    bgz = d[4 * K_NB:5 * K_NB, :]
    nnx = d[5 * K_NB:6 * K_NB, :]
    nny = d[6 * K_NB:7 * K_NB, :]
    nnz = d[7 * K_NB:8 * K_NB, :]

    def finish(w, wgx, wgy, wgz):
        sw = jnp.sum(w, 0, keepdims=True) + EPS_CONST
        swf = jnp.sum(w * f, 0, keepdims=True)
        sgx = jnp.sum(wgx, 0, keepdims=True)
        sgy = jnp.sum(wgy, 0, keepdims=True)
        sgz = jnp.sum(wgz, 0, keepdims=True)
        sfx = jnp.sum(wgx * f, 0, keepdims=True)
        sfy = jnp.sum(wgy * f, 0, keepdims=True)
        sfz = jnp.sum(wgz * f, 0, keepdims=True)
        swx = jnp.sum(w * nnx, 0, keepdims=True)
        swy = jnp.sum(w * nny, 0, keepdims=True)
        swz = jnp.sum(w * nnz, 0, keepdims=True)
        inv = 1.0 / sw
        pot = swf * inv
        gx = (sfx + swx - sgx * pot) * inv
        gy = (sfy + swy - sgy * pot) * inv
        gz = (sfz + swz - sgz * pot) * inv
        return pot, gx, gy, gz

    pot, gx, gy, gz = finish(bw, bgx, bgy, bgz)
    for _ in range(2):
        tx = nnx - gx
        ty = nny - gy
        tz = nnz - gz
        nd2 = tx * tx + ty * ty + tz * tz
        rw = jnp.exp(nd2 * _NEG_INV_SIG)
        pot, gx, gy, gz = finish(bw * rw, bgx * rw, bgy * rw, bgz * rw)

    zeros = jnp.zeros((4, 128), jnp.float32)
    o_ref[0] = jnp.concatenate([pot, gx, gy, gz, zeros], axis=0)


def _tc_reduce(d):
    return pl.pallas_call(
        _tc_body,
        grid=(QG,),
        in_specs=[pl.BlockSpec((1, 8 * K_NB, 128), lambda g: (g, 0, 0))],
        out_specs=pl.BlockSpec((1, 8, 128), lambda g: (g, 0, 0)),
        out_shape=jax.ShapeDtypeStruct((QG, 8, 128), jnp.float32),
    )(d)


@jax.jit
def _run(packed, qx, qy, qz, h, idx2):
    mesh = plsc.VectorSubcoreMesh(core_axis_name="c", subcore_axis_name="s")
    sc_fn = pl.kernel(
        _sc_body,
        out_type=jax.ShapeDtypeStruct((QG, 8 * K_NB, 128), jnp.float32),
        mesh=mesh,
        compiler_params=pltpu.CompilerParams(
            needs_layout_passes=False, use_tc_tiling_on_sc=False),
        scratch_types=(
            pltpu.VMEM((2, NCHUNK, IDX_CHUNK), jnp.int32),
            pltpu.VMEM((2, ROWS_PER_BLK, 8), jnp.float32),
            pltpu.VMEM((2, 8 * K_NB, BLK_Q), jnp.float32),
            pltpu.VMEM((4, QPT), jnp.float32),
            pltpu.SemaphoreType.DMA,
            pltpu.SemaphoreType.DMA,
            pltpu.SemaphoreType.DMA,
        ),
    )
    rows_per_chunk = QC * K_NB // IDX_CHUNK
    outs = []
    for c in range(NCHUNKS_Q):
        lo, hi = c * QC, (c + 1) * QC
        d = sc_fn(packed, qx[lo:hi], qy[lo:hi], qz[lo:hi], h[lo:hi],
                  idx2[c * rows_per_chunk:(c + 1) * rows_per_chunk])
        outs.append(_tc_reduce(d))
    return jnp.concatenate(outs, axis=0)


def kernel(query_points, source_vertices, source_normals, neighbor_indices,
           bandwidth_h, compute_gradient):
    n = source_vertices.shape[0]
    packed = jnp.concatenate(
        [source_vertices, source_normals,
         jnp.zeros((n, 2), jnp.float32)], axis=1)
    idx2 = neighbor_indices.astype(jnp.int32).reshape(-1, IDX_CHUNK)
    res = _run(
        packed, query_points[:, 0], query_points[:, 1], query_points[:, 2],
        bandwidth_h, idx2)
    pot = res[:, 0, :].reshape(Q_TOTAL)
    grad = jnp.stack(
        [res[:, 1, :].reshape(Q_TOTAL),
         res[:, 2, :].reshape(Q_TOTAL),
         res[:, 3, :].reshape(Q_TOTAL)], axis=1)
    grad = jnp.where(compute_gradient != 0, grad, jnp.zeros_like(grad))
    return (pot, grad)


# speedup vs baseline: 1.0836x; 1.0836x over previous
import jax
import jax.numpy as jnp
from jax import lax
from jax.experimental import pallas as pl
from jax.experimental.pallas import tpu as pltpu, tpu_sc as plsc

Q_TOTAL = 65536
N_SRC = 100000
K_NB = 64
SIGMA_N_CONST = 0.8
EPS_CONST = 1e-08
NUM_TILES = 32
QPT = Q_TOTAL // NUM_TILES
BLK_Q = 16
NBLK = QPT // BLK_Q
ROWS_PER_BLK = BLK_Q * K_NB
IDX_CHUNK = 128
NCHUNK = ROWS_PER_BLK // IDX_CHUNK
UNROLL_BASE = 2
UNROLL_REFIT = 4

_NEG_INV_SIG = -1.0 / (SIGMA_N_CONST * SIGMA_N_CONST + EPS_CONST)


def _sc_body(packed_hbm, qx_hbm, qy_hbm, qz_hbm, h_hbm, idx_hbm,
             pot_hbm, gx_hbm, gy_hbm, gz_hbm,
             idx_v, rows_v, der_v, q_v, out_v, gsem, isem):
    nc = 2
    wid = lax.axis_index("s") * nc + lax.axis_index("c")
    qbase = pl.multiple_of(wid * QPT, QPT)

    for i, src_h in enumerate((qx_hbm, qy_hbm, qz_hbm, h_hbm)):
        pltpu.sync_copy(src_h.at[pl.ds(qbase, QPT)], q_v.at[i])

    lane = lax.broadcasted_iota(jnp.int32, (16,), 0)
    lane64 = lane * K_NB
    cols = [jnp.full((16,), c, jnp.int32) for c in range(6)]
    neg_inv_sig = jnp.full((16,), _NEG_INV_SIG, jnp.float32)
    nmax_u = jnp.full((16,), N_SRC - 1, jnp.uint32)

    def idx_slice(b):
        g0 = qbase + b * BLK_Q
        irow = pl.multiple_of(g0 // 2, NCHUNK)
        return idx_hbm.at[pl.ds(irow, NCHUNK)]

    def fire_idx(b, par):
        pltpu.async_copy(idx_slice(b), idx_v.at[par], isem)

    def wait_idx(b, par):
        pltpu.make_async_copy(idx_slice(b), idx_v.at[par], isem).wait()

    def fire_rows(b, par):
        for r in range(NCHUNK):
            for c in range(IDX_CHUNK // 16):
                sl = (par, r, pl.ds(c * 16, 16))
                v = plsc.bitcast(idx_v[sl], jnp.uint32)
                idx_v[sl] = plsc.bitcast(jnp.minimum(v, nmax_u), jnp.int32)
        for j in range(NCHUNK):
            pltpu.async_copy(
                packed_hbm.at[idx_v.at[par, j]],
                rows_v.at[par, pl.ds(j * IDX_CHUNK, IDX_CHUNK)],
                gsem,
            )

    def wait_rows(par):
        for j in range(NCHUNK):
            pltpu.make_async_copy(
                packed_hbm.at[idx_v.at[par, j]],
                rows_v.at[par, pl.ds(j * IDX_CHUNK, IDX_CHUNK)],
                gsem,
            ).wait()

    fire_idx(0, 0)
    wait_idx(0, 0)
    fire_rows(0, 0)
    fire_idx(1, 1)

    def block_body(b, _):
        par = lax.rem(b, 2)
        nxt = 1 - par
        rows_b = rows_v.at[par]

        @pl.when(b + 1 < NBLK)
        def _prefetch_rows():
            wait_idx(b + 1, nxt)
            fire_rows(b + 1, nxt)

        wait_rows(par)

        @pl.when(b + 2 < NBLK)
        def _prefetch_idx():
            fire_idx(b + 2, par)

        qsl = pl.ds(b * BLK_Q, BLK_Q)
        qx = q_v[0, qsl]
        qy = q_v[1, qsl]
        qz = q_v[2, qsl]
        h = q_v[3, qsl]
        neg_inv_h2 = -1.0 / (h * h + EPS_CONST)
        two_inv = 2.0 * neg_inv_h2

        zf = jnp.zeros((16,), jnp.float32)
        acc0 = (zf,) * 11

        def base_k(k, acc):
            (sw, swf, sgx, sgy, sgz, sfx, sfy, sfz, swx, swy, swz) = acc
            row = lane64 + k
            nbx = plsc.load_gather(rows_b, [row, cols[0]])
            nby = plsc.load_gather(rows_b, [row, cols[1]])
            nbz = plsc.load_gather(rows_b, [row, cols[2]])
            nnx = plsc.load_gather(rows_b, [row, cols[3]])
            nny = plsc.load_gather(rows_b, [row, cols[4]])
            nnz = plsc.load_gather(rows_b, [row, cols[5]])
            dx = qx - nbx
            dy = qy - nby
            dz = qz - nbz
            d2 = dx * dx + dy * dy + dz * dz
            bw = jnp.exp(d2 * neg_inv_h2)
            f = dx * nnx + dy * nny + dz * nnz
            s = bw * two_inv
            bgx = s * dx
            bgy = s * dy
            bgz = s * dz
            der_v[0, k, :] = bw
            der_v[1, k, :] = f
            der_v[2, k, :] = bgx
            der_v[3, k, :] = bgy
            der_v[4, k, :] = bgz
            der_v[5, k, :] = nnx
            der_v[6, k, :] = nny
            der_v[7, k, :] = nnz
            return (sw + bw, swf + bw * f,
                    sgx + bgx, sgy + bgy, sgz + bgz,
                    sfx + bgx * f, sfy + bgy * f, sfz + bgz * f,
                    swx + bw * nnx, swy + bw * nny, swz + bw * nnz)

        acc = plsc.parallel_loop(0, K_NB, 1, unroll=UNROLL_BASE, carry=acc0)(base_k)

        def finish(acc):
            (sw, swf, sgx, sgy, sgz, sfx, sfy, sfz, swx, swy, swz) = acc
            swe = sw + EPS_CONST
            inv = 1.0 / swe
            pot = swf * inv
            gx = (sfx + swx - sgx * pot) * inv
            gy = (sfy + swy - sgy * pot) * inv
            gz = (sfz + swz - sgz * pot) * inv
            return pot, gx, gy, gz

        pot, gx, gy, gz = finish(acc)

        for _ in range(2):
            def iter_k(k, acc, gx=gx, gy=gy, gz=gz):
                (sw, swf, sgx, sgy, sgz,
                 sfx, sfy, sfz, swx, swy, swz) = acc
                bw = der_v[0, k, :]
                f = der_v[1, k, :]
                bgx = der_v[2, k, :]
                bgy = der_v[3, k, :]
                bgz = der_v[4, k, :]
                nnx = der_v[5, k, :]
                nny = der_v[6, k, :]
                nnz = der_v[7, k, :]
                tx = nnx - gx
                ty = nny - gy
                tz = nnz - gz
                nd2 = tx * tx + ty * ty + tz * tz
                rw = jnp.exp(nd2 * neg_inv_sig)
                w = bw * rw
                wgx = bgx * rw
                wgy = bgy * rw
                wgz = bgz * rw
                return (sw + w, swf + w * f,
                        sgx + wgx, sgy + wgy, sgz + wgz,
                        sfx + wgx * f, sfy + wgy * f, sfz + wgz * f,
                        swx + w * nnx, swy + w * nny, swz + w * nnz)

            acc = plsc.parallel_loop(0, K_NB, 1, unroll=UNROLL_REFIT, carry=acc0)(iter_k)
            pot, gx, gy, gz = finish(acc)

        out_v[0, qsl] = pot
        out_v[1, qsl] = gx
        out_v[2, qsl] = gy
        out_v[3, qsl] = gz
        return _

    lax.fori_loop(0, NBLK, block_body, 0)

    for i, dst_h in enumerate((pot_hbm, gx_hbm, gy_hbm, gz_hbm)):
        pltpu.sync_copy(out_v.at[i], dst_h.at[pl.ds(qbase, QPT)])


@jax.jit
def _run(packed, qx, qy, qz, h, idx2):
    mesh = plsc.VectorSubcoreMesh(core_axis_name="c", subcore_axis_name="s")
    kfn = pl.kernel(
        _sc_body,
        out_type=tuple(
            jax.ShapeDtypeStruct((Q_TOTAL,), jnp.float32) for _ in range(4)
        ),
        mesh=mesh,
        compiler_params=pltpu.CompilerParams(
            needs_layout_passes=False, use_tc_tiling_on_sc=False),
        scratch_types=(
            pltpu.VMEM((2, NCHUNK, IDX_CHUNK), jnp.int32),
            pltpu.VMEM((2, ROWS_PER_BLK, 8), jnp.float32),
            pltpu.VMEM((8, K_NB, 16), jnp.float32),
            pltpu.VMEM((4, QPT), jnp.float32),
            pltpu.VMEM((4, QPT), jnp.float32),
            pltpu.SemaphoreType.DMA,
            pltpu.SemaphoreType.DMA,
        ),
    )
    return kfn(packed, qx, qy, qz, h, idx2)


def kernel(query_points, source_vertices, source_normals, neighbor_indices,
           bandwidth_h, compute_gradient):
    n = source_vertices.shape[0]
    packed = jnp.concatenate(
        [source_vertices, source_normals,
         jnp.zeros((n, 2), jnp.float32)], axis=1)
    idx2 = neighbor_indices.astype(jnp.int32).reshape(-1, IDX_CHUNK)
    pot, gx, gy, gz = _run(
        packed, query_points[:, 0], query_points[:, 1], query_points[:, 2],
        bandwidth_h, idx2)
    grad = jnp.stack([gx, gy, gz], axis=1)
    grad = jnp.where(compute_gradient != 0, grad, jnp.zeros_like(grad))
    return (pot, grad)


# speedup vs baseline: 1.1183x; 1.0320x over previous
import jax
import jax.numpy as jnp
from jax import lax
from jax.experimental import pallas as pl
from jax.experimental.pallas import tpu as pltpu, tpu_sc as plsc

Q_TOTAL = 65536
N_SRC = 100000
K_NB = 64
SIGMA_N_CONST = 0.8
EPS_CONST = 1e-08
NUM_TILES = 32
QPT = Q_TOTAL // NUM_TILES
BLK_Q = 16
NBLK = QPT // BLK_Q
ROWS_PER_BLK = BLK_Q * K_NB
IDX_CHUNK = 128
NCHUNK = ROWS_PER_BLK // IDX_CHUNK
UNROLL_BASE = 2
UNROLL_REFIT = 2

_NEG_INV_SIG = -1.0 / (SIGMA_N_CONST * SIGMA_N_CONST + EPS_CONST)


def _sc_body(packed_hbm, qx_hbm, qy_hbm, qz_hbm, h_hbm, idx_hbm,
             pot_hbm, gx_hbm, gy_hbm, gz_hbm,
             idx_v, rows_v, der_v, q_v, out_v, gsem, isem):
    nc = 2
    wid = lax.axis_index("s") * nc + lax.axis_index("c")
    qbase = pl.multiple_of(wid * QPT, QPT)

    for i, src_h in enumerate((qx_hbm, qy_hbm, qz_hbm, h_hbm)):
        pltpu.sync_copy(src_h.at[pl.ds(qbase, QPT)], q_v.at[i])

    lane = lax.broadcasted_iota(jnp.int32, (16,), 0)
    lane64 = lane * K_NB
    cols = [jnp.full((16,), c, jnp.int32) for c in range(6)]
    neg_inv_sig = jnp.full((16,), _NEG_INV_SIG, jnp.float32)
    nmax_u = jnp.full((16,), N_SRC - 1, jnp.uint32)

    def idx_slice(b):
        g0 = qbase + b * BLK_Q
        irow = pl.multiple_of(g0 // 2, NCHUNK)
        return idx_hbm.at[pl.ds(irow, NCHUNK)]

    def fire_idx(b, par):
        pltpu.async_copy(idx_slice(b), idx_v.at[par], isem)

    def wait_idx(b, par):
        pltpu.make_async_copy(idx_slice(b), idx_v.at[par], isem).wait()

    def fire_rows(b, par):
        for r in range(NCHUNK):
            for c in range(IDX_CHUNK // 16):
                sl = (par, r, pl.ds(c * 16, 16))
                v = plsc.bitcast(idx_v[sl], jnp.uint32)
                idx_v[sl] = plsc.bitcast(jnp.minimum(v, nmax_u), jnp.int32)
        for j in range(NCHUNK):
            pltpu.async_copy(
                packed_hbm.at[idx_v.at[par, j]],
                rows_v.at[par, pl.ds(j * IDX_CHUNK, IDX_CHUNK)],
                gsem,
            )

    def wait_rows(par):
        for j in range(NCHUNK):
            pltpu.make_async_copy(
                packed_hbm.at[idx_v.at[par, j]],
                rows_v.at[par, pl.ds(j * IDX_CHUNK, IDX_CHUNK)],
                gsem,
            ).wait()

    fire_idx(0, 0)
    wait_idx(0, 0)
    fire_rows(0, 0)
    fire_idx(1, 1)

    def block_body(b, _):
        par = lax.rem(b, 2)
        nxt = 1 - par
        rows_b = rows_v.at[par]

        @pl.when(b + 1 < NBLK)
        def _prefetch_rows():
            wait_idx(b + 1, nxt)
            fire_rows(b + 1, nxt)

        wait_rows(par)

        @pl.when(b + 2 < NBLK)
        def _prefetch_idx():
            fire_idx(b + 2, par)

        qsl = pl.ds(b * BLK_Q, BLK_Q)
        qx = q_v[0, qsl]
        qy = q_v[1, qsl]
        qz = q_v[2, qsl]
        h = q_v[3, qsl]
        neg_inv_h2 = -1.0 / (h * h + EPS_CONST)
        two_inv = 2.0 * neg_inv_h2

        zf = jnp.zeros((16,), jnp.float32)
        acc0 = (zf,) * 11

        def base_k(k, acc):
            (sw, swf, sgx, sgy, sgz, sfx, sfy, sfz, swx, swy, swz) = acc
            row = lane64 + k
            nbx = plsc.load_gather(rows_b, [row, cols[0]])
            nby = plsc.load_gather(rows_b, [row, cols[1]])
            nbz = plsc.load_gather(rows_b, [row, cols[2]])
            nnx = plsc.load_gather(rows_b, [row, cols[3]])
            nny = plsc.load_gather(rows_b, [row, cols[4]])
            nnz = plsc.load_gather(rows_b, [row, cols[5]])
            dx = qx - nbx
            dy = qy - nby
            dz = qz - nbz
            d2 = dx * dx + dy * dy + dz * dz
            bw = jnp.exp(d2 * neg_inv_h2)
            f = dx * nnx + dy * nny + dz * nnz
            s = bw * two_inv
            bgx = s * dx
            bgy = s * dy
            bgz = s * dz
            der_v[0, k, :] = bw
            der_v[1, k, :] = f
            der_v[2, k, :] = bgx
            der_v[3, k, :] = bgy
            der_v[4, k, :] = bgz
            der_v[5, k, :] = nnx
            der_v[6, k, :] = nny
            der_v[7, k, :] = nnz
            return (sw + bw, swf + bw * f,
                    sgx + bgx, sgy + bgy, sgz + bgz,
                    sfx + bgx * f, sfy + bgy * f, sfz + bgz * f,
                    swx + bw * nnx, swy + bw * nny, swz + bw * nnz)

        acc = plsc.parallel_loop(0, K_NB, 1, unroll=UNROLL_BASE, carry=acc0)(base_k)

        def finish(acc):
            (sw, swf, sgx, sgy, sgz, sfx, sfy, sfz, swx, swy, swz) = acc
            swe = sw + EPS_CONST
            inv = 1.0 / swe
            pot = swf * inv
            gx = (sfx + swx - sgx * pot) * inv
            gy = (sfy + swy - sgy * pot) * inv
            gz = (sfz + swz - sgz * pot) * inv
            return pot, gx, gy, gz

        pot, gx, gy, gz = finish(acc)

        for _ in range(2):
            def iter_k(k, acc, gx=gx, gy=gy, gz=gz):
                (sw, swf, sgx, sgy, sgz,
                 sfx, sfy, sfz, swx, swy, swz) = acc
                bw = der_v[0, k, :]
                f = der_v[1, k, :]
                bgx = der_v[2, k, :]
                bgy = der_v[3, k, :]
                bgz = der_v[4, k, :]
                nnx = der_v[5, k, :]
                nny = der_v[6, k, :]
                nnz = der_v[7, k, :]
                tx = nnx - gx
                ty = nny - gy
                tz = nnz - gz
                nd2 = tx * tx + ty * ty + tz * tz
                rw = jnp.exp(nd2 * neg_inv_sig)
                w = bw * rw
                wgx = bgx * rw
                wgy = bgy * rw
                wgz = bgz * rw
                return (sw + w, swf + w * f,
                        sgx + wgx, sgy + wgy, sgz + wgz,
                        sfx + wgx * f, sfy + wgy * f, sfz + wgz * f,
                        swx + w * nnx, swy + w * nny, swz + w * nnz)

            acc = plsc.parallel_loop(0, K_NB, 1, unroll=UNROLL_REFIT, carry=acc0)(iter_k)
            pot, gx, gy, gz = finish(acc)

        out_v[0, qsl] = pot
        out_v[1, qsl] = gx
        out_v[2, qsl] = gy
        out_v[3, qsl] = gz
        return _

    lax.fori_loop(0, NBLK, block_body, 0)

    for i, dst_h in enumerate((pot_hbm, gx_hbm, gy_hbm, gz_hbm)):
        pltpu.sync_copy(out_v.at[i], dst_h.at[pl.ds(qbase, QPT)])


@jax.jit
def _run(packed, qx, qy, qz, h, idx2):
    mesh = plsc.VectorSubcoreMesh(core_axis_name="c", subcore_axis_name="s")
    kfn = pl.kernel(
        _sc_body,
        out_type=tuple(
            jax.ShapeDtypeStruct((Q_TOTAL,), jnp.float32) for _ in range(4)
        ),
        mesh=mesh,
        compiler_params=pltpu.CompilerParams(
            needs_layout_passes=False, use_tc_tiling_on_sc=False),
        scratch_types=(
            pltpu.VMEM((2, NCHUNK, IDX_CHUNK), jnp.int32),
            pltpu.VMEM((2, ROWS_PER_BLK, 8), jnp.float32),
            pltpu.VMEM((8, K_NB, 16), jnp.float32),
            pltpu.VMEM((4, QPT), jnp.float32),
            pltpu.VMEM((4, QPT), jnp.float32),
            pltpu.SemaphoreType.DMA,
            pltpu.SemaphoreType.DMA,
        ),
    )
    return kfn(packed, qx, qy, qz, h, idx2)


def kernel(query_points, source_vertices, source_normals, neighbor_indices,
           bandwidth_h, compute_gradient):
    n = source_vertices.shape[0]
    packed = jnp.concatenate(
        [source_vertices, source_normals,
         jnp.zeros((n, 2), jnp.float32)], axis=1)
    idx2 = neighbor_indices.astype(jnp.int32).reshape(-1, IDX_CHUNK)
    pot, gx, gy, gz = _run(
        packed, query_points[:, 0], query_points[:, 1], query_points[:, 2],
        bandwidth_h, idx2)
    grad = jnp.stack([gx, gy, gz], axis=1)
    grad = jnp.where(compute_gradient != 0, grad, jnp.zeros_like(grad))
    return (pot, grad)
